# baseline (speedup 1.0000x reference)
import functools

import jax
import jax.numpy as jnp
from jax.experimental import pallas as pl
from jax.experimental.pallas import tpu as pltpu


def _chamfer_body(x1_ref, x2t_ref, o_ref, *, mc, scale1, scale2):
    b = pl.program_id(0)

    a = x1_ref[0]
    bt = x2t_ref[0]
    m = bt.shape[1]
    n_chunks = m // mc

    sq1 = jnp.sum(a * a, axis=1, keepdims=True)
    a2 = -2.0 * a

    @pl.when(b == 0)
    def _():
        o_ref[0, 0] = 0.0

    rowmin = None
    colsum = jnp.float32(0.0)
    for c in range(n_chunks):
        btc = bt[:, c * mc:(c + 1) * mc]
        sq2c = jnp.sum(btc * btc, axis=0, keepdims=True)
        inner2 = jax.lax.dot_general(
            a2, btc, (((1,), (0,)), ((), ())),
            preferred_element_type=jnp.float32,
        )
        d = (sq1 + sq2c) + inner2
        rm_c = jnp.min(d, axis=1)
        rowmin = rm_c if rowmin is None else jnp.minimum(rowmin, rm_c)
        colsum += jnp.sum(jnp.maximum(jnp.min(d, axis=0), 0.0))

    rowsum = jnp.sum(jnp.maximum(rowmin, 0.0))
    o_ref[0, 0] += rowsum * scale1 + colsum * scale2


@jax.jit
def kernel(xyz1, xyz2):
    B, N, _ = xyz1.shape
    _, M, _ = xyz2.shape

    x2t = jnp.transpose(xyz2, (0, 2, 1))

    body = functools.partial(
        _chamfer_body,
        mc=512,
        scale1=1.0 / (B * N),
        scale2=1.0 / (B * M),
    )

    out = pl.pallas_call(
        body,
        grid=(B,),
        in_specs=[
            pl.BlockSpec((1, N, 3), lambda b: (b, 0, 0)),
            pl.BlockSpec((1, 3, M), lambda b: (b, 0, 0)),
        ],
        out_specs=pl.BlockSpec(
            (1, 1), lambda b: (0, 0), memory_space=pltpu.SMEM
        ),
        out_shape=jax.ShapeDtypeStruct((1, 1), jnp.float32),
    )(xyz1, x2t)

    return out[0, 0]


# speedup vs baseline: 1.0182x; 1.0182x over previous
import functools

import jax
import jax.numpy as jnp
from jax.experimental import pallas as pl
from jax.experimental.pallas import tpu as pltpu


def _chamfer_body(x1_ref, x2t_ref, o_ref, *, mc, scale1, scale2):
    b = pl.program_id(0)

    a = x1_ref[0]
    bt = x2t_ref[0]
    m = bt.shape[1]
    n_chunks = m // mc

    sq1 = jnp.sum(a * a, axis=1, keepdims=True)
    a2 = -2.0 * a

    @pl.when(b == 0)
    def _():
        o_ref[0, 0] = 0.0

    rowmin = None
    colsum = jnp.float32(0.0)
    for c in range(n_chunks):
        btc = bt[:, c * mc:(c + 1) * mc]
        sq2c = jnp.sum(btc * btc, axis=0, keepdims=True)
        inner2 = jax.lax.dot_general(
            a2, btc, (((1,), (0,)), ((), ())),
            preferred_element_type=jnp.float32,
        )
        d = (sq1 + sq2c) + inner2
        rm_c = jnp.min(d, axis=1)
        rowmin = rm_c if rowmin is None else jnp.minimum(rowmin, rm_c)
        colsum += jnp.sum(jnp.maximum(jnp.min(d, axis=0), 0.0))

    rowsum = jnp.sum(jnp.maximum(rowmin, 0.0))
    o_ref[0, 0] += rowsum * scale1 + colsum * scale2


@jax.jit
def kernel(xyz1, xyz2):
    B, N, _ = xyz1.shape
    _, M, _ = xyz2.shape

    x2t = jnp.transpose(xyz2, (0, 2, 1))

    body = functools.partial(
        _chamfer_body,
        mc=2048,
        scale1=1.0 / (B * N),
        scale2=1.0 / (B * M),
    )

    out = pl.pallas_call(
        body,
        grid=(B,),
        in_specs=[
            pl.BlockSpec((1, N, 3), lambda b: (b, 0, 0)),
            pl.BlockSpec((1, 3, M), lambda b: (b, 0, 0)),
        ],
        out_specs=pl.BlockSpec(
            (1, 1), lambda b: (0, 0), memory_space=pltpu.SMEM
        ),
        out_shape=jax.ShapeDtypeStruct((1, 1), jnp.float32),
    )(xyz1, x2t)

    return out[0, 0]
